# baseline (speedup 1.0000x reference)
import functools

import jax
import jax.numpy as jnp
from jax import lax
from jax.experimental import pallas as pl
from jax.experimental.pallas import tpu as pltpu
from jax.experimental.pallas import tpu_sc as plsc

_N = 10000
_D = 128
_NC = 2
_NS = 16
_NW = _NC * _NS
_CH = 128
_NPAD = 10048
_RPT = 632
_LRPT = _NPAD - 15 * _RPT
_DW = 128


def _rows_of(s):
    return s * _RPT


def _make_segsum(k_chunks: int):
    assert k_chunks % 2 == 0
    mesh = plsc.VectorSubcoreMesh(core_axis_name="c", subcore_axis_name="s")
    out_type = [jax.ShapeDtypeStruct((_NC, _NPAD, _D), jnp.float32)]
    scratch = [
        pltpu.VMEM((k_chunks, _CH), jnp.int32),
        pltpu.VMEM((2, _CH), jnp.int32),
        pltpu.VMEM((2, _CH, _D), jnp.float32),
        pltpu.VMEM_SHARED((_NPAD, _D), jnp.float32),
        pltpu.SemaphoreType.DMA,
        pltpu.SemaphoreType.DMA,
    ]

    def body(x_hbm, src_hbm, dst_hbm, zmsg_hbm, msg_out,
             src_v, dstb_v, rows_v, acc_sh, sem0, sem1):
        c = lax.axis_index("c")
        s = lax.axis_index("s")
        t = c * _NS + s
        r0 = s * _RPT

        pltpu.sync_copy(src_hbm.at[t], src_v)

        @pl.when(s < _NS - 1)
        def _():
            pltpu.sync_copy(zmsg_hbm.at[pl.ds(r0, _RPT)],
                            acc_sh.at[pl.ds(r0, _RPT)])

        @pl.when(s == _NS - 1)
        def _():
            pltpu.sync_copy(zmsg_hbm.at[pl.ds(r0, _LRPT)],
                            acc_sh.at[pl.ds(r0, _LRPT)])
        plsc.subcore_barrier()

        def _start(k, slot, sem):
            pltpu.async_copy(x_hbm.at[src_v.at[k]], rows_v.at[slot], sem)
            pltpu.async_copy(dst_hbm.at[t, k], dstb_v.at[slot], sem)

        def _wait(k, slot, sem):
            pltpu.make_async_copy(x_hbm.at[src_v.at[k]], rows_v.at[slot],
                                  sem).wait()
            pltpu.make_async_copy(dst_hbm.at[t, k], dstb_v.at[slot],
                                  sem).wait()

        def _scat(k, slot):
            pltpu.sync_copy(rows_v.at[slot], acc_sh.at[dstb_v.at[slot]],
                            add=True)

        half = k_chunks // 2
        _start(0, 0, sem0)

        def _loop(kk, carry):
            k0 = kk * 2
            _start(k0 + 1, 1, sem1)
            _wait(k0, 0, sem0)
            _scat(k0, 0)

            @pl.when(kk < half - 1)
            def _():
                _start(k0 + 2, 0, sem0)
            _wait(k0 + 1, 1, sem1)
            _scat(k0 + 1, 1)
            return carry
        lax.fori_loop(0, half, _loop, 0)

        plsc.subcore_barrier()

        @pl.when(s < _NS - 1)
        def _():
            pltpu.sync_copy(acc_sh.at[pl.ds(r0, _RPT)],
                            msg_out.at[c, pl.ds(r0, _RPT)])

        @pl.when(s == _NS - 1)
        def _():
            pltpu.sync_copy(acc_sh.at[pl.ds(r0, _LRPT)],
                            msg_out.at[c, pl.ds(r0, _LRPT)])

    return pl.kernel(body, out_type=out_type, mesh=mesh,
                     scratch_types=scratch)


def _make_segdeg(k_chunks: int):
    mesh = plsc.VectorSubcoreMesh(core_axis_name="c", subcore_axis_name="s")
    out_type = [jax.ShapeDtypeStruct((_NC, _NPAD, _DW), jnp.float32)]
    scratch = [
        pltpu.VMEM((k_chunks, _CH), jnp.int32),
        pltpu.VMEM((_CH, _DW), jnp.float32),
        pltpu.VMEM_SHARED((_NPAD, _DW), jnp.float32),
    ]

    def body(dst_hbm, zdeg_hbm, ones_hbm, deg_out, dst_v, ones_v, dacc_sh):
        c = lax.axis_index("c")
        s = lax.axis_index("s")
        t = c * _NS + s
        r0 = s * _RPT
        pltpu.sync_copy(dst_hbm.at[t], dst_v)
        pltpu.sync_copy(ones_hbm, ones_v)

        @pl.when(s < _NS - 1)
        def _():
            pltpu.sync_copy(zdeg_hbm.at[pl.ds(r0, _RPT)],
                            dacc_sh.at[pl.ds(r0, _RPT)])

        @pl.when(s == _NS - 1)
        def _():
            pltpu.sync_copy(zdeg_hbm.at[pl.ds(r0, _LRPT)],
                            dacc_sh.at[pl.ds(r0, _LRPT)])
        plsc.subcore_barrier()

        def _loop(k, carry):
            pltpu.sync_copy(ones_v, dacc_sh.at[dst_v.at[k]], add=True)
            return carry
        lax.fori_loop(0, k_chunks, _loop, 0)

        plsc.subcore_barrier()

        @pl.when(s < _NS - 1)
        def _():
            pltpu.sync_copy(dacc_sh.at[pl.ds(r0, _RPT)],
                            deg_out.at[c, pl.ds(r0, _RPT)])

        @pl.when(s == _NS - 1)
        def _():
            pltpu.sync_copy(dacc_sh.at[pl.ds(r0, _LRPT)],
                            deg_out.at[c, pl.ds(r0, _LRPT)])

    return pl.kernel(body, out_type=out_type, mesh=mesh,
                     scratch_types=scratch)


_BLK = 1000


def _tc1_body(msgp, x, degp, w1, b1, h1_out):
    deg = (degp[0] + degp[1])[:, :1] + 1.0
    agg = (msgp[0] + msgp[1] + x[...]) / deg
    h = jnp.dot(agg, w1[...], preferred_element_type=jnp.float32) + b1[...]
    h1_out[...] = jnp.maximum(h, 0.0)


def _tc2_body(msgp, h1, degp, w2, b2, wp1, bp1, wp2, bp2, wc1, bc1, wc2, bc2,
              z_out, c_out):
    deg = (degp[0] + degp[1])[:, :1] + 1.0
    agg = (msgp[0] + msgp[1] + h1[...]) / deg
    h2 = jnp.dot(agg, w2[...], preferred_element_type=jnp.float32) + b2[...]
    u = jnp.maximum(jnp.dot(h2, wp1[...], preferred_element_type=jnp.float32)
                    + bp1[...], 0.0)
    zp = jnp.dot(u, wp2[...], preferred_element_type=jnp.float32) + bp2[...]
    nrm = jnp.sqrt(jnp.sum(zp * zp, axis=1, keepdims=True))
    z_out[...] = zp / jnp.maximum(nrm, 1e-12)
    v = jnp.maximum(jnp.dot(h2, wc1[...], preferred_element_type=jnp.float32)
                    + bc1[...], 0.0)
    lg = jnp.dot(v, wc2[...], preferred_element_type=jnp.float32) + bc2[...]
    m = jnp.max(lg, axis=1, keepdims=True)
    e = jnp.exp(lg - m)
    c_out[...] = e / jnp.sum(e, axis=1, keepdims=True)


def _full(arr):
    return pl.BlockSpec(arr.shape, lambda i: (0,) * arr.ndim)


def _tc1(msgp, x, degp, w1, b1):
    grid = (_N // _BLK,)
    return pl.pallas_call(
        _tc1_body,
        grid=grid,
        in_specs=[
            pl.BlockSpec((_NC, _BLK, _D), lambda i: (0, i, 0)),
            pl.BlockSpec((_BLK, _D), lambda i: (i, 0)),
            pl.BlockSpec((_NC, _BLK, _DW), lambda i: (0, i, 0)),
            _full(w1), _full(b1),
        ],
        out_specs=pl.BlockSpec((_BLK, _D), lambda i: (i, 0)),
        out_shape=jax.ShapeDtypeStruct((_N, _D), jnp.float32),
    )(msgp, x, degp, w1, b1)


def _tc2(msgp, h1, degp, w2, b2, wp1, bp1, wp2, bp2, wc1, bc1, wc2, bc2):
    grid = (_N // _BLK,)
    dp = wp2.shape[1]
    ncl = wc2.shape[1]
    return pl.pallas_call(
        _tc2_body,
        grid=grid,
        in_specs=[
            pl.BlockSpec((_NC, _BLK, _D), lambda i: (0, i, 0)),
            pl.BlockSpec((_BLK, _D), lambda i: (i, 0)),
            pl.BlockSpec((_NC, _BLK, _DW), lambda i: (0, i, 0)),
            _full(w2), _full(b2), _full(wp1), _full(bp1), _full(wp2),
            _full(bp2), _full(wc1), _full(bc1), _full(wc2), _full(bc2),
        ],
        out_specs=[
            pl.BlockSpec((_BLK, dp), lambda i: (i, 0)),
            pl.BlockSpec((_BLK, ncl), lambda i: (i, 0)),
        ],
        out_shape=[
            jax.ShapeDtypeStruct((_N, dp), jnp.float32),
            jax.ShapeDtypeStruct((_N, ncl), jnp.float32),
        ],
    )(msgp, h1, degp, w2, b2, wp1, bp1, wp2, bp2, wc1, bc1, wc2, bc2)


def _prep_edges(ei):
    e = ei.shape[1]
    k = -(-e // (_NW * _CH))
    k += k % 2
    epad = _NW * k * _CH
    src = ei[0].astype(jnp.int32)
    dst = ei[1].astype(jnp.int32)
    pad = jnp.arange(epad - e, dtype=jnp.int32)
    src = jnp.concatenate([src, pad % _N])
    dst = jnp.concatenate([dst, _N + pad % (_NPAD - _N)])
    return src.reshape(_NW, k, _CH), dst.reshape(_NW, k, _CH), k


def kernel(x_i, edge_index_i, x_j, edge_index_j, W1, b1, W2, b2,
           Wp1, bp1, Wp2, bp2, Wc1, bc1, Wc2, bc2):
    src_i, dst_i, k = _prep_edges(edge_index_i)
    src_j, dst_j, _ = _prep_edges(edge_index_j)
    zm = jnp.zeros((_NPAD, _D), jnp.float32)
    ones_pay = jnp.ones((_CH, _DW), jnp.float32)
    seg = _make_segsum(k)
    segdeg = _make_segdeg(k)
    b1r, b2r = b1.reshape(1, _D), b2.reshape(1, _D)
    bp1r, bp2r = bp1.reshape(1, -1), bp2.reshape(1, -1)
    bc1r, bc2r = bc1.reshape(1, -1), bc2.reshape(1, -1)

    (msgp_i,) = seg(x_i, src_i, dst_i, zm)
    (msgp_j,) = seg(x_j, src_j, dst_j, zm)
    (degp_i,) = segdeg(dst_i, zm, ones_pay)
    (degp_j,) = segdeg(dst_j, zm, ones_pay)
    h1_i = _tc1(msgp_i, x_i, degp_i, W1, b1r)
    h1_j = _tc1(msgp_j, x_j, degp_j, W1, b1r)
    (msg2_i,) = seg(h1_i, src_i, dst_i, zm)
    (msg2_j,) = seg(h1_j, src_j, dst_j, zm)
    z_i, c_i = _tc2(msg2_i, h1_i, degp_i, W2, b2r, Wp1, bp1r, Wp2, bp2r,
                    Wc1, bc1r, Wc2, bc2r)
    z_j, c_j = _tc2(msg2_j, h1_j, degp_j, W2, b2r, Wp1, bp1r, Wp2, bp2r,
                    Wc1, bc1r, Wc2, bc2r)
    return (z_i, z_j, c_i, c_j)


# speedup vs baseline: 1.0081x; 1.0081x over previous
import functools

import jax
import jax.numpy as jnp
from jax import lax
from jax.experimental import pallas as pl
from jax.experimental.pallas import tpu as pltpu
from jax.experimental.pallas import tpu_sc as plsc

_N = 10000
_D = 128
_NC = 2
_NS = 16
_NW = _NC * _NS
_CH = 128
_NPAD = 10048
_RPT = 632
_LRPT = _NPAD - 15 * _RPT
_DW = 128


def _rows_of(s):
    return s * _RPT


def _make_segsum(k_chunks: int):
    assert k_chunks % 2 == 0
    mesh = plsc.VectorSubcoreMesh(core_axis_name="c", subcore_axis_name="s")
    out_type = [jax.ShapeDtypeStruct((_NC, _NPAD, _D), jnp.float32)]
    scratch = [
        pltpu.VMEM((k_chunks, _CH), jnp.int32),
        pltpu.VMEM((2, _CH), jnp.int32),
        pltpu.VMEM((2, _CH, _D), jnp.float32),
        pltpu.VMEM_SHARED((_NPAD, _D), jnp.float32),
        pltpu.SemaphoreType.DMA,
        pltpu.SemaphoreType.DMA,
    ]

    def body(x_hbm, src_hbm, dst_hbm, zmsg_hbm, msg_out,
             src_v, dstb_v, rows_v, acc_sh, sem0, sem1):
        c = lax.axis_index("c")
        s = lax.axis_index("s")
        t = c * _NS + s
        r0 = s * _RPT

        pltpu.sync_copy(src_hbm.at[t], src_v)

        @pl.when(s < _NS - 1)
        def _():
            pltpu.sync_copy(zmsg_hbm.at[pl.ds(r0, _RPT)],
                            acc_sh.at[pl.ds(r0, _RPT)])

        @pl.when(s == _NS - 1)
        def _():
            pltpu.sync_copy(zmsg_hbm.at[pl.ds(r0, _LRPT)],
                            acc_sh.at[pl.ds(r0, _LRPT)])
        plsc.subcore_barrier()

        def _start(k, slot, sem):
            pltpu.async_copy(x_hbm.at[src_v.at[k]], rows_v.at[slot], sem)
            pltpu.async_copy(dst_hbm.at[t, k], dstb_v.at[slot], sem)

        def _wait(k, slot, sem):
            pltpu.make_async_copy(x_hbm.at[src_v.at[k]], rows_v.at[slot],
                                  sem).wait()
            pltpu.make_async_copy(dst_hbm.at[t, k], dstb_v.at[slot],
                                  sem).wait()

        def _scat(k, slot):
            pltpu.sync_copy(rows_v.at[slot], acc_sh.at[dstb_v.at[slot]],
                            add=True)

        half = k_chunks // 2
        _start(0, 0, sem0)

        def _loop(kk, carry):
            k0 = kk * 2
            _start(k0 + 1, 1, sem1)
            _wait(k0, 0, sem0)
            _scat(k0, 0)

            @pl.when(kk < half - 1)
            def _():
                _start(k0 + 2, 0, sem0)
            _wait(k0 + 1, 1, sem1)
            _scat(k0 + 1, 1)
            return carry
        lax.fori_loop(0, half, _loop, 0)

        plsc.subcore_barrier()

        @pl.when(s < _NS - 1)
        def _():
            pltpu.sync_copy(acc_sh.at[pl.ds(r0, _RPT)],
                            msg_out.at[c, pl.ds(r0, _RPT)])

        @pl.when(s == _NS - 1)
        def _():
            pltpu.sync_copy(acc_sh.at[pl.ds(r0, _LRPT)],
                            msg_out.at[c, pl.ds(r0, _LRPT)])

    return pl.kernel(body, out_type=out_type, mesh=mesh,
                     scratch_types=scratch)


def _make_segdeg(k_chunks: int):
    mesh = plsc.VectorSubcoreMesh(core_axis_name="c", subcore_axis_name="s")
    out_type = [jax.ShapeDtypeStruct((_NC, _NPAD, _DW), jnp.float32)]
    scratch = [
        pltpu.VMEM((k_chunks, _CH), jnp.int32),
        pltpu.VMEM((_CH, _DW), jnp.float32),
        pltpu.VMEM_SHARED((_NPAD, _DW), jnp.float32),
    ]

    def body(dst_hbm, zdeg_hbm, ones_hbm, deg_out, dst_v, ones_v, dacc_sh):
        c = lax.axis_index("c")
        s = lax.axis_index("s")
        t = c * _NS + s
        r0 = s * _RPT
        pltpu.sync_copy(dst_hbm.at[t], dst_v)
        pltpu.sync_copy(ones_hbm, ones_v)

        @pl.when(s < _NS - 1)
        def _():
            pltpu.sync_copy(zdeg_hbm.at[pl.ds(r0, _RPT)],
                            dacc_sh.at[pl.ds(r0, _RPT)])

        @pl.when(s == _NS - 1)
        def _():
            pltpu.sync_copy(zdeg_hbm.at[pl.ds(r0, _LRPT)],
                            dacc_sh.at[pl.ds(r0, _LRPT)])
        plsc.subcore_barrier()

        def _loop(k, carry):
            pltpu.sync_copy(ones_v, dacc_sh.at[dst_v.at[k]], add=True)
            return carry
        lax.fori_loop(0, k_chunks, _loop, 0)

        plsc.subcore_barrier()

        @pl.when(s < _NS - 1)
        def _():
            pltpu.sync_copy(dacc_sh.at[pl.ds(r0, _RPT)],
                            deg_out.at[c, pl.ds(r0, _RPT)])

        @pl.when(s == _NS - 1)
        def _():
            pltpu.sync_copy(dacc_sh.at[pl.ds(r0, _LRPT)],
                            deg_out.at[c, pl.ds(r0, _LRPT)])

    return pl.kernel(body, out_type=out_type, mesh=mesh,
                     scratch_types=scratch)


_BLK = 1000


def _tc1_body(msgp, x, degp, w1, b1, h1_out):
    deg = (degp[0] + degp[1])[:, :1] + 1.0
    agg = (msgp[0] + msgp[1] + x[...]) / deg
    h = jnp.dot(agg, w1[...], preferred_element_type=jnp.float32) + b1[...]
    h1_out[...] = jnp.maximum(h, 0.0)


def _tc2_body(msgp, h1, degp, w2, b2, wp1, bp1, wp2, bp2, wc1, bc1, wc2, bc2,
              z_out, c_out):
    deg = (degp[0] + degp[1])[:, :1] + 1.0
    agg = (msgp[0] + msgp[1] + h1[...]) / deg
    h2 = jnp.dot(agg, w2[...], preferred_element_type=jnp.float32) + b2[...]
    u = jnp.maximum(jnp.dot(h2, wp1[...], preferred_element_type=jnp.float32)
                    + bp1[...], 0.0)
    zp = jnp.dot(u, wp2[...], preferred_element_type=jnp.float32) + bp2[...]
    nrm = jnp.sqrt(jnp.sum(zp * zp, axis=1, keepdims=True))
    z_out[...] = zp / jnp.maximum(nrm, 1e-12)
    v = jnp.maximum(jnp.dot(h2, wc1[...], preferred_element_type=jnp.float32)
                    + bc1[...], 0.0)
    lg = jnp.dot(v, wc2[...], preferred_element_type=jnp.float32) + bc2[...]
    m = jnp.max(lg, axis=1, keepdims=True)
    e = jnp.exp(lg - m)
    c_out[...] = e / jnp.sum(e, axis=1, keepdims=True)


def _full(arr):
    return pl.BlockSpec(arr.shape, lambda i: (0,) * arr.ndim)


def _tc1(msgp, x, degp, w1, b1):
    grid = (_N // _BLK,)
    return pl.pallas_call(
        _tc1_body,
        grid=grid,
        in_specs=[
            pl.BlockSpec((_NC, _BLK, _D), lambda i: (0, i, 0)),
            pl.BlockSpec((_BLK, _D), lambda i: (i, 0)),
            pl.BlockSpec((_NC, _BLK, _DW), lambda i: (0, i, 0)),
            _full(w1), _full(b1),
        ],
        out_specs=pl.BlockSpec((_BLK, _D), lambda i: (i, 0)),
        out_shape=jax.ShapeDtypeStruct((_N, _D), jnp.float32),
    )(msgp, x, degp, w1, b1)


def _tc2(msgp, h1, degp, w2, b2, wp1, bp1, wp2, bp2, wc1, bc1, wc2, bc2):
    grid = (_N // _BLK,)
    dp = wp2.shape[1]
    ncl = wc2.shape[1]
    return pl.pallas_call(
        _tc2_body,
        grid=grid,
        in_specs=[
            pl.BlockSpec((_NC, _BLK, _D), lambda i: (0, i, 0)),
            pl.BlockSpec((_BLK, _D), lambda i: (i, 0)),
            pl.BlockSpec((_NC, _BLK, _DW), lambda i: (0, i, 0)),
            _full(w2), _full(b2), _full(wp1), _full(bp1), _full(wp2),
            _full(bp2), _full(wc1), _full(bc1), _full(wc2), _full(bc2),
        ],
        out_specs=[
            pl.BlockSpec((_BLK, dp), lambda i: (i, 0)),
            pl.BlockSpec((_BLK, ncl), lambda i: (i, 0)),
        ],
        out_shape=[
            jax.ShapeDtypeStruct((_N, dp), jnp.float32),
            jax.ShapeDtypeStruct((_N, ncl), jnp.float32),
        ],
    )(msgp, h1, degp, w2, b2, wp1, bp1, wp2, bp2, wc1, bc1, wc2, bc2)


def _prep_edges(ei):
    e = ei.shape[1]
    k = -(-e // (_NW * _CH))
    k += k % 2
    epad = _NW * k * _CH
    src = ei[0].astype(jnp.int32)
    dst = ei[1].astype(jnp.int32)
    pad_src = jnp.arange(epad - e, dtype=jnp.int32) % _N
    src = jnp.concatenate([src, pad_src])
    dst = jnp.concatenate([dst, jnp.full((epad - e,), _N, jnp.int32)])
    return src.reshape(_NW, k, _CH), dst.reshape(_NW, k, _CH), k


def kernel(x_i, edge_index_i, x_j, edge_index_j, W1, b1, W2, b2,
           Wp1, bp1, Wp2, bp2, Wc1, bc1, Wc2, bc2):
    src_i, dst_i, k = _prep_edges(edge_index_i)
    src_j, dst_j, _ = _prep_edges(edge_index_j)
    zm = jnp.zeros((_NPAD, _D), jnp.float32)
    ones_pay = jnp.ones((_CH, _DW), jnp.float32)
    seg = _make_segsum(k)
    segdeg = _make_segdeg(k)
    b1r, b2r = b1.reshape(1, _D), b2.reshape(1, _D)
    bp1r, bp2r = bp1.reshape(1, -1), bp2.reshape(1, -1)
    bc1r, bc2r = bc1.reshape(1, -1), bc2.reshape(1, -1)

    (msgp_i,) = seg(x_i, src_i, dst_i, zm)
    (msgp_j,) = seg(x_j, src_j, dst_j, zm)
    (degp_i,) = segdeg(dst_i, zm, ones_pay)
    (degp_j,) = segdeg(dst_j, zm, ones_pay)
    h1_i = _tc1(msgp_i, x_i, degp_i, W1, b1r)
    h1_j = _tc1(msgp_j, x_j, degp_j, W1, b1r)
    (msg2_i,) = seg(h1_i, src_i, dst_i, zm)
    (msg2_j,) = seg(h1_j, src_j, dst_j, zm)
    z_i, c_i = _tc2(msg2_i, h1_i, degp_i, W2, b2r, Wp1, bp1r, Wp2, bp2r,
                    Wc1, bc1r, Wc2, bc2r)
    z_j, c_j = _tc2(msg2_j, h1_j, degp_j, W2, b2r, Wp1, bp1r, Wp2, bp2r,
                    Wc1, bc1r, Wc2, bc2r)
    return (z_i, z_j, c_i, c_j)
